# Initial kernel scaffold; baseline (speedup 1.0000x reference)
import jax
import jax.numpy as jnp
from jax.experimental import pallas as pl


def kernel(x, edge_index, batch_vec, W1, b1, W2, b2, W3, b3, W4, b4):
    raise NotImplementedError("write your pallas kernel here")


# speedup vs baseline: 41.9472x; 41.9472x over previous
import functools

import jax
import jax.numpy as jnp
from jax import lax
from jax.experimental import pallas as pl
from jax.experimental.pallas import tpu as pltpu
from jax.experimental.pallas import tpu_sc as plsc

N = 10000
E = 320000
F_IN = 128
H = 16
LIN = 100
NG = 128
NC = 12

NCORES = 2
NSUB = 16
NW = NCORES * NSUB

N_PAD = 10240
ROWS_PER_SUB = N_PAD // NSUB
GROUPS_PER_SUB = ROWS_PER_SUB // H
CH = 128
CHUNKS = 80
E_TILE = CH * CHUNKS
E_PAD = E_TILE * NW
E_ROWS = E_PAD // CH
POOL_ROWS = NG + 8
POOL_FLAT = POOL_ROWS * H
NBUF = 10

_mesh = plsc.VectorSubcoreMesh(
    core_axis_name="c", subcore_axis_name="s", num_cores=NCORES,
    num_subcores=NSUB)


def _f32(*shape):
    return jax.ShapeDtypeStruct(shape, jnp.float32)


@functools.partial(
    pl.kernel,
    out_type=_f32(NCORES * N_PAD),
    mesh=_mesh,
    scratch_types=[
        pltpu.VMEM((CHUNKS, CH), jnp.int32),
        pltpu.VMEM((CH,), jnp.float32),
        pltpu.VMEM_SHARED((N_PAD,), jnp.float32),
        pltpu.SemaphoreType.DMA,
    ],
)
def _sc_degree(dst_hbm, zeros_hbm, deg_out, didx, ones_v, deg_sh, sem):
    c = lax.axis_index("c")
    s = lax.axis_index("s")
    wid = c * NSUB + s
    pltpu.sync_copy(zeros_hbm.at[pl.ds(0, ROWS_PER_SUB)],
                    deg_sh.at[pl.ds(s * ROWS_PER_SUB, ROWS_PER_SUB)])
    one = jnp.ones((H,), jnp.float32)
    for k in range(CH // H):
        ones_v[pl.ds(k * H, H)] = one
    pltpu.sync_copy(dst_hbm.at[pl.ds(wid * CHUNKS, CHUNKS)], didx)
    plsc.subcore_barrier()

    DEPTH = 8

    @pl.loop(0, CHUNKS // DEPTH)
    def _(t):
        sc = [pltpu.async_copy(ones_v, deg_sh.at[didx.at[t * DEPTH + b]],
                               sem, add=True) for b in range(DEPTH)]
        for b in range(DEPTH):
            sc[b].wait()
    plsc.subcore_barrier()
    pltpu.sync_copy(deg_sh.at[pl.ds(s * ROWS_PER_SUB, ROWS_PER_SUB)],
                    deg_out.at[pl.ds(c * N_PAD + s * ROWS_PER_SUB,
                                     ROWS_PER_SUB)])


def _edge_loop(src_hbm, dst_hbm, table_hbm, zeros_hbm,
               sidx, didx, rows, acc_sh, gsem, ssem, s, wid):
    pltpu.sync_copy(zeros_hbm,
                    acc_sh.at[pl.ds(s * ROWS_PER_SUB, ROWS_PER_SUB)])
    pltpu.sync_copy(src_hbm.at[pl.ds(wid * CHUNKS, CHUNKS)], sidx)
    pltpu.sync_copy(dst_hbm.at[pl.ds(wid * CHUNKS, CHUNKS)], didx)
    plsc.subcore_barrier()

    @pl.loop(0, CHUNKS // NBUF)
    def _(t):
        j0 = t * NBUF
        g = [pltpu.async_copy(table_hbm.at[sidx.at[j0 + b]], rows.at[b],
                              gsem.at[b]) for b in range(NBUF)]
        sc = []
        for b in range(NBUF):
            g[b].wait()
            sc.append(pltpu.async_copy(rows.at[b], acc_sh.at[didx.at[j0 + b]],
                                       ssem.at[b], add=True))
        for b in range(NBUF):
            sc[b].wait()
    plsc.subcore_barrier()


_EDGE_SCRATCH = [
    pltpu.VMEM((CHUNKS, CH), jnp.int32),
    pltpu.VMEM((CHUNKS, CH), jnp.int32),
    pltpu.VMEM((NBUF, CH, H), jnp.float32),
    pltpu.VMEM_SHARED((N_PAD, H), jnp.float32),
    pltpu.SemaphoreType.DMA((NBUF,)),
    pltpu.SemaphoreType.DMA((NBUF,)),
]


@functools.partial(
    pl.kernel,
    out_type=_f32(NCORES, N_PAD, H),
    mesh=_mesh,
    compiler_params=pltpu.CompilerParams(use_tc_tiling_on_sc=False),
    scratch_types=_EDGE_SCRATCH,
)
def _sc_edge_acc(src_hbm, dst_hbm, table_hbm, zeros_hbm, acc_out,
                 sidx, didx, rows, acc_sh, gsem, ssem):
    c = lax.axis_index("c")
    s = lax.axis_index("s")
    wid = c * NSUB + s
    _edge_loop(src_hbm, dst_hbm, table_hbm, zeros_hbm,
               sidx, didx, rows, acc_sh, gsem, ssem, s, wid)
    pltpu.sync_copy(acc_sh.at[pl.ds(s * ROWS_PER_SUB, ROWS_PER_SUB)],
                    acc_out.at[c, pl.ds(s * ROWS_PER_SUB, ROWS_PER_SUB)])


@functools.partial(
    pl.kernel,
    out_type=_f32(NW * POOL_FLAT),
    mesh=_mesh,
    compiler_params=pltpu.CompilerParams(use_tc_tiling_on_sc=False,
                                         needs_layout_passes=False),
    scratch_types=_EDGE_SCRATCH + [
        pltpu.VMEM((ROWS_PER_SUB, H), jnp.float32),
        pltpu.VMEM((ROWS_PER_SUB, H), jnp.float32),
        pltpu.VMEM((ROWS_PER_SUB,), jnp.float32),
        pltpu.VMEM((ROWS_PER_SUB,), jnp.int32),
        pltpu.VMEM((H,), jnp.float32),
        pltpu.VMEM((POOL_FLAT,), jnp.float32),
    ],
)
def _sc_edge_pool(src_hbm, dst_hbm, table_hbm, zeros_hbm, dinv_hbm, seg_hbm,
                  b2_hbm, zp_hbm, pool_out,
                  sidx, didx, rows, acc_sh, gsem, ssem,
                  accv, zv, dv, sv, b2v, poolv):
    c = lax.axis_index("c")
    s = lax.axis_index("s")
    wid = c * NSUB + s
    _edge_loop(src_hbm, dst_hbm, table_hbm, zeros_hbm,
               sidx, didx, rows, acc_sh, gsem, ssem, s, wid)

    nbase = s * ROWS_PER_SUB
    pltpu.sync_copy(acc_sh.at[pl.ds(nbase, ROWS_PER_SUB)], accv)
    pltpu.sync_copy(table_hbm.at[pl.ds(nbase, ROWS_PER_SUB)], zv)
    pltpu.sync_copy(dinv_hbm.at[pl.ds(nbase, ROWS_PER_SUB)], dv)
    pltpu.sync_copy(seg_hbm.at[pl.ds(nbase, ROWS_PER_SUB)], sv)
    pltpu.sync_copy(b2_hbm, b2v)
    pltpu.sync_copy(zp_hbm, poolv)
    zc = jnp.where(c == 0, 1.0, 0.0)
    b2r = b2v[...] * zc
    lanes = lax.iota(jnp.int32, H)

    @pl.loop(0, GROUPS_PER_SUB)
    def _(g):
        dvec = dv[pl.ds(g * H, H)]
        svec = sv[pl.ds(g * H, H)]
        for k in range(H):
            i = g * H + k
            v = dvec[k] * (accv[i] + zc * zv[i]) + b2r
            idx = svec[k] * H + lanes
            plsc.addupdate_scatter(poolv, [idx], v)

    pltpu.sync_copy(poolv, pool_out.at[pl.ds(wid * POOL_FLAT, POOL_FLAT)])


def _tc_xw_body(x_ref, w1_ref, xw_ref):
    xw = jnp.dot(x_ref[...], w1_ref[...], preferred_element_type=jnp.float32)
    xw_ref[...] = jnp.concatenate(
        [xw, jnp.zeros((N_PAD - N, H), jnp.float32)], axis=0)


def _tc_scale_body(xw_ref, deg_ref, y_ref, dinv_ref):
    deg = deg_ref[0] + deg_ref[1] + 1.0
    dinv = lax.rsqrt(deg)
    dinv_ref[...] = dinv
    y_ref[...] = dinv[:, None] * xw_ref[...]


def _tc_mid_body(acc_ref, y_ref, dinv_ref, b1_ref, w2_ref, z_ref):
    agg = acc_ref[0] + acc_ref[1] + y_ref[...]
    dinv = dinv_ref[...]
    h = jax.nn.relu(dinv[:, None] * agg + b1_ref[...])
    z_ref[...] = jnp.dot(dinv[:, None] * h, w2_ref[...],
                         preferred_element_type=jnp.float32)


def _tc_head_body(pool_ref, w3_ref, b3_ref, w4_ref, b4_ref, out_ref):
    g = jnp.sum(pool_ref[...], axis=0)[:NG]
    g = jax.nn.relu(g)
    t = jax.nn.relu(jnp.dot(g, w3_ref[...],
                            preferred_element_type=jnp.float32) + b3_ref[...])
    out_ref[...] = jnp.dot(t, w4_ref[...],
                           preferred_element_type=jnp.float32) + b4_ref[...]


def kernel(x, edge_index, batch_vec, W1, b1, W2, b2, W3, b3, W4, b4):
    src = edge_index[0]
    dst = edge_index[1]
    pad_e = E_PAD - E
    src_p = jnp.concatenate([src, jnp.zeros((pad_e,), jnp.int32)])
    dst_p = jnp.concatenate([dst, jnp.full((pad_e,), N, jnp.int32)])
    src2d = src_p.reshape(E_ROWS, CH)
    dst2d = dst_p.reshape(E_ROWS, CH)
    batch_p = jnp.concatenate(
        [batch_vec, jnp.full((N_PAD - N,), NG, jnp.int32)])
    zeros_rows = jnp.zeros((ROWS_PER_SUB, H), jnp.float32)
    zeros_deg = jnp.zeros((N_PAD,), jnp.float32)
    zeros_pool = jnp.zeros((POOL_FLAT,), jnp.float32)

    deg2 = _sc_degree(dst2d, zeros_deg).reshape(NCORES, N_PAD)
    xw1 = pl.pallas_call(_tc_xw_body, out_shape=_f32(N_PAD, H))(x, W1)

    y1, dinv = pl.pallas_call(
        _tc_scale_body,
        out_shape=(_f32(N_PAD, H), _f32(N_PAD)),
    )(xw1, deg2)

    acc1 = _sc_edge_acc(src2d, dst2d, y1, zeros_rows)

    z = pl.pallas_call(
        _tc_mid_body,
        out_shape=_f32(N_PAD, H),
    )(acc1, y1, dinv, b1, W2)

    pools = _sc_edge_pool(src2d, dst2d, z, zeros_rows, dinv, batch_p, b2,
                          zeros_pool)

    out = pl.pallas_call(
        _tc_head_body,
        out_shape=_f32(NG, NC),
    )(pools.reshape(NW, POOL_ROWS, H), W3, b3, W4, b4)
    return out


# speedup vs baseline: 70.0500x; 1.6700x over previous
import functools

import jax
import jax.numpy as jnp
from jax import lax
from jax.experimental import pallas as pl
from jax.experimental.pallas import tpu as pltpu
from jax.experimental.pallas import tpu_sc as plsc

N = 10000
E = 320000
F_IN = 128
H = 16
LIN = 100
NG = 128
NC = 12

NCORES = 2
NSUB = 16
NW = NCORES * NSUB

N_PAD = 10240
ROWS_PER_SUB = N_PAD // NSUB
GROUPS_PER_SUB = ROWS_PER_SUB // H
CH = 128
CHUNKS = 80
E_TILE = CH * CHUNKS
E_PAD = E_TILE * NW
E_ROWS = E_PAD // CH
POOL_ROWS = NG + 8
POOL_FLAT = POOL_ROWS * H
NBUF = 10

_mesh = plsc.VectorSubcoreMesh(
    core_axis_name="c", subcore_axis_name="s", num_cores=NCORES,
    num_subcores=NSUB)


def _f32(*shape):
    return jax.ShapeDtypeStruct(shape, jnp.float32)


@functools.partial(
    pl.kernel,
    out_type=_f32(NCORES * N_PAD),
    mesh=_mesh,
    scratch_types=[
        pltpu.VMEM((CHUNKS, CH), jnp.int32),
        pltpu.VMEM((CH,), jnp.float32),
        pltpu.VMEM_SHARED((N_PAD,), jnp.float32),
        pltpu.SemaphoreType.DMA,
    ],
)
def _sc_degree(dst_hbm, zeros_hbm, deg_out, didx, ones_v, deg_sh, sem):
    c = lax.axis_index("c")
    s = lax.axis_index("s")
    wid = c * NSUB + s
    pltpu.sync_copy(zeros_hbm.at[pl.ds(0, ROWS_PER_SUB)],
                    deg_sh.at[pl.ds(s * ROWS_PER_SUB, ROWS_PER_SUB)])
    one = jnp.ones((H,), jnp.float32)
    for k in range(CH // H):
        ones_v[pl.ds(k * H, H)] = one
    pltpu.sync_copy(dst_hbm.at[pl.ds(wid * CHUNKS, CHUNKS)], didx)
    plsc.subcore_barrier()

    DEPTH = 8

    @pl.loop(0, CHUNKS // DEPTH)
    def _(t):
        sc = [pltpu.async_copy(ones_v, deg_sh.at[didx.at[t * DEPTH + b]],
                               sem, add=True) for b in range(DEPTH)]
        for b in range(DEPTH):
            sc[b].wait()
    plsc.subcore_barrier()
    pltpu.sync_copy(deg_sh.at[pl.ds(s * ROWS_PER_SUB, ROWS_PER_SUB)],
                    deg_out.at[pl.ds(c * N_PAD + s * ROWS_PER_SUB,
                                     ROWS_PER_SUB)])


def _edge_loop(src_hbm, dst_hbm, table_hbm, zeros_hbm,
               sidx, didx, rows, acc_sh, gsem, ssem, s, wid):
    pltpu.sync_copy(zeros_hbm,
                    acc_sh.at[pl.ds(s * ROWS_PER_SUB, ROWS_PER_SUB)])
    pltpu.sync_copy(src_hbm.at[pl.ds(wid * CHUNKS, CHUNKS)], sidx)
    pltpu.sync_copy(dst_hbm.at[pl.ds(wid * CHUNKS, CHUNKS)], didx)
    plsc.subcore_barrier()

    @pl.loop(0, CHUNKS // NBUF)
    def _(t):
        j0 = t * NBUF
        g = [pltpu.async_copy(table_hbm.at[sidx.at[j0 + b]], rows.at[b],
                              gsem.at[b]) for b in range(NBUF)]
        sc = []
        for b in range(NBUF):
            g[b].wait()
            sc.append(pltpu.async_copy(rows.at[b], acc_sh.at[didx.at[j0 + b]],
                                       ssem.at[b], add=True))
        for b in range(NBUF):
            sc[b].wait()
    plsc.subcore_barrier()


_EDGE_SCRATCH = [
    pltpu.VMEM((CHUNKS, CH), jnp.int32),
    pltpu.VMEM((CHUNKS, CH), jnp.int32),
    pltpu.VMEM((NBUF, CH, H), jnp.float32),
    pltpu.VMEM_SHARED((N_PAD, H), jnp.float32),
    pltpu.SemaphoreType.DMA((NBUF,)),
    pltpu.SemaphoreType.DMA((NBUF,)),
]


@functools.partial(
    pl.kernel,
    out_type=_f32(NCORES, N_PAD, H),
    mesh=_mesh,
    compiler_params=pltpu.CompilerParams(use_tc_tiling_on_sc=False),
    scratch_types=_EDGE_SCRATCH,
)
def _sc_edge_acc(src_hbm, dst_hbm, table_hbm, zeros_hbm, acc_out,
                 sidx, didx, rows, acc_sh, gsem, ssem):
    c = lax.axis_index("c")
    s = lax.axis_index("s")
    wid = c * NSUB + s
    _edge_loop(src_hbm, dst_hbm, table_hbm, zeros_hbm,
               sidx, didx, rows, acc_sh, gsem, ssem, s, wid)
    pltpu.sync_copy(acc_sh.at[pl.ds(s * ROWS_PER_SUB, ROWS_PER_SUB)],
                    acc_out.at[c, pl.ds(s * ROWS_PER_SUB, ROWS_PER_SUB)])


@functools.partial(
    pl.kernel,
    out_type=_f32(NW * POOL_FLAT),
    mesh=_mesh,
    compiler_params=pltpu.CompilerParams(use_tc_tiling_on_sc=False,
                                         needs_layout_passes=False),
    scratch_types=_EDGE_SCRATCH + [
        pltpu.VMEM((ROWS_PER_SUB, H), jnp.float32),
        pltpu.VMEM((ROWS_PER_SUB, H), jnp.float32),
        pltpu.VMEM((ROWS_PER_SUB,), jnp.float32),
        pltpu.VMEM((ROWS_PER_SUB,), jnp.int32),
        pltpu.VMEM((H,), jnp.float32),
        pltpu.VMEM((POOL_FLAT,), jnp.float32),
    ],
)
def _sc_edge_pool(src_hbm, dst_hbm, table_hbm, zeros_hbm, dinv_hbm, seg_hbm,
                  b2_hbm, zp_hbm, pool_out,
                  sidx, didx, rows, acc_sh, gsem, ssem,
                  accv, zv, dv, sv, b2v, poolv):
    c = lax.axis_index("c")
    s = lax.axis_index("s")
    wid = c * NSUB + s
    _edge_loop(src_hbm, dst_hbm, table_hbm, zeros_hbm,
               sidx, didx, rows, acc_sh, gsem, ssem, s, wid)

    nbase = s * ROWS_PER_SUB
    pltpu.sync_copy(acc_sh.at[pl.ds(nbase, ROWS_PER_SUB)], accv)
    pltpu.sync_copy(table_hbm.at[pl.ds(nbase, ROWS_PER_SUB)], zv)
    pltpu.sync_copy(dinv_hbm.at[pl.ds(nbase, ROWS_PER_SUB)], dv)
    pltpu.sync_copy(seg_hbm.at[pl.ds(nbase, ROWS_PER_SUB)], sv)
    pltpu.sync_copy(b2_hbm, b2v)
    pltpu.sync_copy(zp_hbm, poolv)
    zc = jnp.where(c == 0, 1.0, 0.0)
    b2r = b2v[...] * zc
    lanes = lax.iota(jnp.int32, H)

    @pl.loop(0, GROUPS_PER_SUB)
    def _(g):
        dvec = dv[pl.ds(g * H, H)]
        svec = sv[pl.ds(g * H, H)]
        for k in range(H):
            i = g * H + k
            v = dvec[k] * (accv[i] + zc * zv[i]) + b2r
            idx = svec[k] * H + lanes
            plsc.addupdate_scatter(poolv, [idx], v)

    pltpu.sync_copy(poolv, pool_out.at[pl.ds(wid * POOL_FLAT, POOL_FLAT)])


def _tc_xw_body(x_ref, w1_ref, xw_ref):
    xw = jnp.dot(x_ref[...], w1_ref[...], preferred_element_type=jnp.float32)
    xw_ref[...] = jnp.concatenate(
        [xw, jnp.zeros((N_PAD - N, H), jnp.float32)], axis=0)


def _tc_scale_body(xw_ref, deg_ref, y_ref, dinv_ref):
    deg = deg_ref[0] + deg_ref[1] + 1.0
    dinv = lax.rsqrt(deg)
    dinv_ref[...] = dinv
    y_ref[...] = dinv[:, None] * xw_ref[...]


def _tc_mid_body(acc_ref, y_ref, dinv_ref, b1_ref, w2_ref, z_ref):
    agg = acc_ref[0] + acc_ref[1] + y_ref[...]
    dinv = dinv_ref[...]
    h = jax.nn.relu(dinv[:, None] * agg + b1_ref[...])
    z_ref[...] = jnp.dot(dinv[:, None] * h, w2_ref[...],
                         preferred_element_type=jnp.float32)


def _tc_head_body(pool_ref, w3_ref, b3_ref, w4_ref, b4_ref, out_ref):
    g = jnp.sum(pool_ref[...], axis=0)[:NG]
    g = jax.nn.relu(g)
    t = jax.nn.relu(jnp.dot(g, w3_ref[...],
                            preferred_element_type=jnp.float32) + b3_ref[...])
    out_ref[...] = jnp.dot(t, w4_ref[...],
                           preferred_element_type=jnp.float32) + b4_ref[...]


def kernel(x, edge_index, batch_vec, W1, b1, W2, b2, W3, b3, W4, b4):
    src = edge_index[0]
    dst = edge_index[1]
    pad_e = E_PAD - E
    pidx = jnp.arange(pad_e, dtype=jnp.int32)
    src_p = jnp.concatenate([src, (pidx * 53) % N])
    dst_p = jnp.concatenate([dst, N + pidx % (N_PAD - N)])
    src2d = src_p.reshape(E_ROWS, CH)
    dst2d = dst_p.reshape(E_ROWS, CH)
    batch_p = jnp.concatenate(
        [batch_vec, jnp.full((N_PAD - N,), NG, jnp.int32)])
    zeros_rows = jnp.zeros((ROWS_PER_SUB, H), jnp.float32)
    zeros_deg = jnp.zeros((N_PAD,), jnp.float32)
    zeros_pool = jnp.zeros((POOL_FLAT,), jnp.float32)

    deg2 = _sc_degree(dst2d, zeros_deg).reshape(NCORES, N_PAD)
    xw1 = pl.pallas_call(_tc_xw_body, out_shape=_f32(N_PAD, H))(x, W1)

    y1, dinv = pl.pallas_call(
        _tc_scale_body,
        out_shape=(_f32(N_PAD, H), _f32(N_PAD)),
    )(xw1, deg2)

    acc1 = _sc_edge_acc(src2d, dst2d, y1, zeros_rows)

    z = pl.pallas_call(
        _tc_mid_body,
        out_shape=_f32(N_PAD, H),
    )(acc1, y1, dinv, b1, W2)

    pools = _sc_edge_pool(src2d, dst2d, z, zeros_rows, dinv, batch_p, b2,
                          zeros_pool)

    out = pl.pallas_call(
        _tc_head_body,
        out_shape=_f32(NG, NC),
    )(pools.reshape(NW, POOL_ROWS, H), W3, b3, W4, b4)
    return out


# speedup vs baseline: 73.6003x; 1.0507x over previous
import functools

import jax
import jax.numpy as jnp
from jax import lax
from jax.experimental import pallas as pl
from jax.experimental.pallas import tpu as pltpu
from jax.experimental.pallas import tpu_sc as plsc

N = 10000
E = 320000
F_IN = 128
H = 16
LIN = 100
NG = 128
NC = 12

NCORES = 2
NSUB = 16
NW = NCORES * NSUB

N_PAD = 10240
ROWS_PER_SUB = N_PAD // NSUB
GROUPS_PER_SUB = ROWS_PER_SUB // H
CH = 128
E_TILE = E // NW
CHUNKS = E_TILE // CH
TAIL = E_TILE - CHUNKS * CH
POOL_ROWS = NG + 8
POOL_FLAT = POOL_ROWS * H
NBUF = 8
EPI = CHUNKS - (CHUNKS // NBUF) * NBUF

_mesh = plsc.VectorSubcoreMesh(
    core_axis_name="c", subcore_axis_name="s", num_cores=NCORES,
    num_subcores=NSUB)


def _f32(*shape):
    return jax.ShapeDtypeStruct(shape, jnp.float32)


@functools.partial(
    pl.kernel,
    out_type=_f32(NCORES * N_PAD),
    mesh=_mesh,
    compiler_params=pltpu.CompilerParams(use_tc_tiling_on_sc=False),
    scratch_types=[
        pltpu.VMEM((E_TILE,), jnp.int32),
        pltpu.VMEM((CH,), jnp.float32),
        pltpu.VMEM_SHARED((N_PAD,), jnp.float32),
        pltpu.SemaphoreType.DMA,
    ],
)
def _sc_degree(ei_hbm, zeros_hbm, deg_out, didx, ones_v, deg_sh, sem):
    c = lax.axis_index("c")
    s = lax.axis_index("s")
    wid = c * NSUB + s
    pltpu.sync_copy(zeros_hbm.at[pl.ds(0, ROWS_PER_SUB)],
                    deg_sh.at[pl.ds(s * ROWS_PER_SUB, ROWS_PER_SUB)])
    one = jnp.ones((H,), jnp.float32)
    for k in range(CH // H):
        ones_v[pl.ds(k * H, H)] = one
    pltpu.sync_copy(ei_hbm.at[1, pl.ds(wid * E_TILE, E_TILE)], didx)
    plsc.subcore_barrier()

    DEPTH = 6

    @pl.loop(0, CHUNKS // DEPTH)
    def _(t):
        sc = [pltpu.async_copy(
            ones_v, deg_sh.at[didx.at[pl.ds((t * DEPTH + b) * CH, CH)]],
            sem, add=True) for b in range(DEPTH)]
        for b in range(DEPTH):
            sc[b].wait()
    pltpu.sync_copy(ones_v.at[pl.ds(0, TAIL)],
                    deg_sh.at[didx.at[pl.ds(CHUNKS * CH, TAIL)]], add=True)
    plsc.subcore_barrier()
    pltpu.sync_copy(deg_sh.at[pl.ds(s * ROWS_PER_SUB, ROWS_PER_SUB)],
                    deg_out.at[pl.ds(c * N_PAD + s * ROWS_PER_SUB,
                                     ROWS_PER_SUB)])


def _edge_loop(ei_hbm, table_hbm, zeros_hbm,
               sidx, didx, rows, acc_sh, gsem, ssem, s, wid):
    pltpu.sync_copy(zeros_hbm,
                    acc_sh.at[pl.ds(s * ROWS_PER_SUB, ROWS_PER_SUB)])
    pltpu.sync_copy(ei_hbm.at[0, pl.ds(wid * E_TILE, E_TILE)], sidx)
    pltpu.sync_copy(ei_hbm.at[1, pl.ds(wid * E_TILE, E_TILE)], didx)
    plsc.subcore_barrier()

    @pl.loop(0, CHUNKS // NBUF)
    def _(t):
        j0 = t * NBUF * CH
        g = [pltpu.async_copy(table_hbm.at[sidx.at[pl.ds(j0 + b * CH, CH)]],
                              rows.at[b], gsem.at[b]) for b in range(NBUF)]
        sc = []
        for b in range(NBUF):
            g[b].wait()
            sc.append(pltpu.async_copy(
                rows.at[b], acc_sh.at[didx.at[pl.ds(j0 + b * CH, CH)]],
                ssem.at[b], add=True))
        for b in range(NBUF):
            sc[b].wait()
    j0 = (CHUNKS // NBUF) * NBUF * CH
    g = [pltpu.async_copy(table_hbm.at[sidx.at[pl.ds(j0 + b * CH, CH)]],
                          rows.at[b], gsem.at[b]) for b in range(EPI)]
    sc = []
    for b in range(EPI):
        g[b].wait()
        sc.append(pltpu.async_copy(
            rows.at[b], acc_sh.at[didx.at[pl.ds(j0 + b * CH, CH)]],
            ssem.at[b], add=True))
    for b in range(EPI):
        sc[b].wait()
    pltpu.async_copy(table_hbm.at[sidx.at[pl.ds(CHUNKS * CH, TAIL)]],
                     rows.at[0, pl.ds(0, TAIL)], gsem.at[0]).wait()
    pltpu.sync_copy(rows.at[0, pl.ds(0, TAIL)],
                    acc_sh.at[didx.at[pl.ds(CHUNKS * CH, TAIL)]], add=True)
    plsc.subcore_barrier()


_EDGE_SCRATCH = [
    pltpu.VMEM((E_TILE,), jnp.int32),
    pltpu.VMEM((E_TILE,), jnp.int32),
    pltpu.VMEM((NBUF, CH, H), jnp.float32),
    pltpu.VMEM_SHARED((N_PAD, H), jnp.float32),
    pltpu.SemaphoreType.DMA((NBUF,)),
    pltpu.SemaphoreType.DMA((NBUF,)),
]


@functools.partial(
    pl.kernel,
    out_type=_f32(NCORES, N_PAD, H),
    mesh=_mesh,
    compiler_params=pltpu.CompilerParams(use_tc_tiling_on_sc=False),
    scratch_types=_EDGE_SCRATCH,
)
def _sc_edge_acc(ei_hbm, table_hbm, zeros_hbm, acc_out,
                 sidx, didx, rows, acc_sh, gsem, ssem):
    c = lax.axis_index("c")
    s = lax.axis_index("s")
    wid = c * NSUB + s
    _edge_loop(ei_hbm, table_hbm, zeros_hbm,
               sidx, didx, rows, acc_sh, gsem, ssem, s, wid)
    pltpu.sync_copy(acc_sh.at[pl.ds(s * ROWS_PER_SUB, ROWS_PER_SUB)],
                    acc_out.at[c, pl.ds(s * ROWS_PER_SUB, ROWS_PER_SUB)])


@functools.partial(
    pl.kernel,
    out_type=_f32(NW * POOL_FLAT),
    mesh=_mesh,
    compiler_params=pltpu.CompilerParams(use_tc_tiling_on_sc=False,
                                         needs_layout_passes=False),
    scratch_types=_EDGE_SCRATCH + [
        pltpu.VMEM((ROWS_PER_SUB, H), jnp.float32),
        pltpu.VMEM((ROWS_PER_SUB, H), jnp.float32),
        pltpu.VMEM((ROWS_PER_SUB,), jnp.float32),
        pltpu.VMEM((ROWS_PER_SUB,), jnp.int32),
        pltpu.VMEM((H,), jnp.float32),
        pltpu.VMEM((POOL_FLAT,), jnp.float32),
    ],
)
def _sc_edge_pool(ei_hbm, table_hbm, zeros_hbm, dinv_hbm, seg_hbm,
                  b2_hbm, zp_hbm, pool_out,
                  sidx, didx, rows, acc_sh, gsem, ssem,
                  accv, zv, dv, sv, b2v, poolv):
    c = lax.axis_index("c")
    s = lax.axis_index("s")
    wid = c * NSUB + s
    _edge_loop(ei_hbm, table_hbm, zeros_hbm,
               sidx, didx, rows, acc_sh, gsem, ssem, s, wid)

    nbase = s * ROWS_PER_SUB
    pltpu.sync_copy(acc_sh.at[pl.ds(nbase, ROWS_PER_SUB)], accv)
    pltpu.sync_copy(table_hbm.at[pl.ds(nbase, ROWS_PER_SUB)], zv)
    pltpu.sync_copy(dinv_hbm.at[pl.ds(nbase, ROWS_PER_SUB)], dv)
    pltpu.sync_copy(seg_hbm.at[pl.ds(nbase, ROWS_PER_SUB)], sv)
    pltpu.sync_copy(b2_hbm, b2v)
    pltpu.sync_copy(zp_hbm, poolv)
    zc = jnp.where(c == 0, 1.0, 0.0)
    b2r = b2v[...] * zc
    lanes = lax.iota(jnp.int32, H)

    @pl.loop(0, GROUPS_PER_SUB)
    def _(g):
        dvec = dv[pl.ds(g * H, H)]
        svec = sv[pl.ds(g * H, H)]
        for k in range(H):
            i = g * H + k
            v = dvec[k] * (accv[i] + zc * zv[i]) + b2r
            idx = svec[k] * H + lanes
            plsc.addupdate_scatter(poolv, [idx], v)

    pltpu.sync_copy(poolv, pool_out.at[pl.ds(wid * POOL_FLAT, POOL_FLAT)])


def _tc_xw_body(x_ref, w1_ref, xw_ref):
    xw = jnp.dot(x_ref[...], w1_ref[...], preferred_element_type=jnp.float32)
    xw_ref[...] = jnp.concatenate(
        [xw, jnp.zeros((N_PAD - N, H), jnp.float32)], axis=0)


def _tc_scale_body(xw_ref, deg_ref, y_ref, dinv_ref):
    deg = deg_ref[0] + deg_ref[1] + 1.0
    dinv = lax.rsqrt(deg)
    dinv_ref[...] = dinv
    y_ref[...] = dinv[:, None] * xw_ref[...]


def _tc_mid_body(acc_ref, y_ref, dinv_ref, b1_ref, w2_ref, z_ref):
    agg = acc_ref[0] + acc_ref[1] + y_ref[...]
    dinv = dinv_ref[...]
    h = jax.nn.relu(dinv[:, None] * agg + b1_ref[...])
    z_ref[...] = jnp.dot(dinv[:, None] * h, w2_ref[...],
                         preferred_element_type=jnp.float32)


def _tc_head_body(pool_ref, w3_ref, b3_ref, w4_ref, b4_ref, out_ref):
    g = jnp.sum(pool_ref[...], axis=0)[:NG]
    g = jax.nn.relu(g)
    t = jax.nn.relu(jnp.dot(g, w3_ref[...],
                            preferred_element_type=jnp.float32) + b3_ref[...])
    out_ref[...] = jnp.dot(t, w4_ref[...],
                           preferred_element_type=jnp.float32) + b4_ref[...]


def kernel(x, edge_index, batch_vec, W1, b1, W2, b2, W3, b3, W4, b4):
    batch_p = jnp.concatenate(
        [batch_vec, jnp.full((N_PAD - N,), NG, jnp.int32)])
    zeros_rows = jnp.zeros((ROWS_PER_SUB, H), jnp.float32)
    zeros_deg = jnp.zeros((N_PAD,), jnp.float32)
    zeros_pool = jnp.zeros((POOL_FLAT,), jnp.float32)

    deg2 = _sc_degree(edge_index, zeros_deg).reshape(NCORES, N_PAD)
    xw1 = pl.pallas_call(_tc_xw_body, out_shape=_f32(N_PAD, H))(x, W1)

    y1, dinv = pl.pallas_call(
        _tc_scale_body,
        out_shape=(_f32(N_PAD, H), _f32(N_PAD)),
    )(xw1, deg2)

    acc1 = _sc_edge_acc(edge_index, y1, zeros_rows)

    z = pl.pallas_call(
        _tc_mid_body,
        out_shape=_f32(N_PAD, H),
    )(acc1, y1, dinv, b1, W2)

    pools = _sc_edge_pool(edge_index, z, zeros_rows, dinv, batch_p, b2,
                          zeros_pool)

    out = pl.pallas_call(
        _tc_head_body,
        out_shape=_f32(NG, NC),
    )(pools.reshape(NW, POOL_ROWS, H), W3, b3, W4, b4)
    return out
